# Initial kernel scaffold; baseline (speedup 1.0000x reference)
import jax
import jax.numpy as jnp
from jax.experimental import pallas as pl


def kernel(x, adj, memb, w_slab, sample_key):
    raise NotImplementedError("write your pallas kernel here")


# speedup vs baseline: 3.7716x; 3.7716x over previous
import jax
import jax.numpy as jnp
from jax.experimental import pallas as pl
from jax.experimental.pallas import tpu as pltpu

_B = 2
_S = 16
_I = 4
_FIN = 8
_H = 32
_HEADS = 4
_G = 4
_NGR = _G * _B
_ROWS = _NGR * _S
_PROWS = _NGR * _I
_LANES = 128

_R_W1 = 0
_R_W2 = 8
_R_AD1 = 40
_R_AS1 = 48
_R_AD2 = 56
_R_AS2 = 64
_R_GB = 72
_R_WH1 = 80
_R_WH1B = 112
_R_WH2 = 120
_R_WH2B = 128
_WBLK = 136


def _gat_layer(feat, w_all, a_dst_rows, a_src_rows, bias, mask):
    hf = jnp.dot(feat, w_all, preferred_element_type=jnp.float32)
    ed = jax.lax.dot_general(hf, a_dst_rows,
                             dimension_numbers=(((1,), (1,)), ((), ())),
                             preferred_element_type=jnp.float32)
    es = jax.lax.dot_general(a_src_rows, hf,
                             dimension_numbers=(((1,), (1,)), ((), ())),
                             preferred_element_type=jnp.float32)

    acc = None
    for h in range(_HEADS):
        e = ed[:, h:h + 1] + es[h:h + 1, :]
        e = jnp.where(e > 0, e, 0.2 * e) + mask
        e = e - jnp.max(e, axis=-1, keepdims=True)
        p = jnp.exp(e)
        denom = jnp.maximum(jnp.sum(p, axis=-1, keepdims=True), 1e-30)
        alpha = p / denom
        part = jnp.dot(alpha, hf[:, h * _H:(h + 1) * _H],
                       preferred_element_type=jnp.float32)
        acc = part if acc is None else acc + part
    return acc * (1.0 / _HEADS) + bias


def _fused_kernel(x_ref, adj_ref, memb_ref, w_ref, out_ref):
    f32 = jnp.float32
    x = x_ref[0]
    adjc = adj_ref[0]
    membc = memb_ref[0]

    w1 = w_ref[_R_W1:_R_W1 + _FIN, :]
    w2 = w_ref[_R_W2:_R_W2 + _H, :]
    ad1 = w_ref[_R_AD1:_R_AD1 + _HEADS, :]
    as1 = w_ref[_R_AS1:_R_AS1 + _HEADS, :]
    ad2 = w_ref[_R_AD2:_R_AD2 + _HEADS, :]
    as2 = w_ref[_R_AS2:_R_AS2 + _HEADS, :]
    b1 = w_ref[_R_GB:_R_GB + 1, 0:_H]
    b2 = w_ref[_R_GB + 1:_R_GB + 2, 0:_H]
    wh1_w = w_ref[_R_WH1:_R_WH1 + _H, 0:2 * _H]
    wh1_b = w_ref[_R_WH1B:_R_WH1B + 1, 0:2 * _H]
    wh2_wt = w_ref[_R_WH2:_R_WH2 + 2, 0:2 * _H]
    wh2_b = w_ref[_R_WH2B:_R_WH2B + 1, 0:2]

    adj_t = jnp.concatenate([adjc] * _NGR, axis=1)
    row_g = jax.lax.broadcasted_iota(jnp.int32, (_ROWS, _ROWS), 0) // _S
    lane_g = jax.lax.broadcasted_iota(jnp.int32, (_ROWS, _ROWS), 1) // _S
    mask = jnp.where((row_g == lane_g) & (adj_t > 0), 0.0, -1e9)

    h1 = jnp.maximum(_gat_layer(x, w1, ad1, as1, b1, mask), 0.0)
    h2 = jnp.maximum(_gat_layer(h1, w2, ad2, as2, b2, mask), 0.0)

    cnt = jnp.maximum(jnp.sum(membc, axis=-1, keepdims=True), 1.0)
    memb_t = jnp.concatenate([membc / cnt] * _NGR, axis=1)
    prow_g = jax.lax.broadcasted_iota(jnp.int32, (_PROWS, _ROWS), 0) // _I
    plane_g = jax.lax.broadcasted_iota(jnp.int32, (_PROWS, _ROWS), 1) // _S
    memb_bd = jnp.where(prow_g == plane_g, memb_t, 0.0)
    sp = jnp.dot(memb_bd, h2, preferred_element_type=f32)

    hid = jnp.maximum(jnp.dot(sp, wh1_w, preferred_element_type=f32) + wh1_b, 0.0)
    lv = jax.lax.dot_general(hid, wh2_wt,
                             dimension_numbers=(((1,), (1,)), ((), ())),
                             preferred_element_type=f32) + wh2_b

    logits_c = lv[:, 0:1]
    values_c = lv[:, 1:2]
    z = logits_c - jnp.max(logits_c)
    ez = jnp.exp(z)
    gr = jax.lax.broadcasted_iota(jnp.int32, (_PROWS, _PROWS), 0) // _I
    gc = jax.lax.broadcasted_iota(jnp.int32, (_PROWS, _PROWS), 1) // _I
    grp = (gr == gc).astype(f32)
    ssum = jnp.dot(grp, ez, preferred_element_type=f32)
    logp = z - jnp.log(ssum)
    pr = jnp.exp(logp)
    ent = -jnp.dot(grp, pr * logp, preferred_element_type=f32)

    lane = jax.lax.broadcasted_iota(jnp.int32, (_PROWS, 8), 1)
    out = (jnp.where(lane == 0, logits_c, 0.0)
           + jnp.where(lane == 1, values_c, 0.0)
           + jnp.where(lane == 2, logp, 0.0)
           + jnp.where(lane == 3, ent, 0.0))
    out_ref[0] = out.astype(out_ref.dtype)


def kernel(x, adj, memb, w_slab, sample_key):
    n = x.shape[0]
    ng = -(-n // _G)
    pad = ng * _G - n
    if pad:
        x = jnp.concatenate([x, jnp.zeros((pad,) + x.shape[1:], x.dtype)], 0)
        eye = jnp.broadcast_to(jnp.eye(_S, dtype=adj.dtype), (pad, _B, _S, _S))
        adj = jnp.concatenate([adj, eye], 0)
        memb = jnp.concatenate([memb, jnp.zeros((pad,) + memb.shape[1:], memb.dtype)], 0)

    xs = x.reshape(ng, _ROWS, _FIN)
    adjc = adj.reshape(ng, _ROWS, _S)
    membc = memb.reshape(ng, _PROWS, _S)

    out = pl.pallas_call(
        _fused_kernel,
        out_shape=jax.ShapeDtypeStruct((ng, _PROWS, 8), jnp.float32),
        grid=(ng,),
        in_specs=[
            pl.BlockSpec((1, _ROWS, _FIN), lambda g: (g, 0, 0)),
            pl.BlockSpec((1, _ROWS, _S), lambda g: (g, 0, 0)),
            pl.BlockSpec((1, _PROWS, _S), lambda g: (g, 0, 0)),
            pl.BlockSpec((_WBLK, _LANES), lambda g: (0, 0)),
        ],
        out_specs=pl.BlockSpec((1, _PROWS, 8), lambda g: (g, 0, 0)),
        compiler_params=pltpu.CompilerParams(dimension_semantics=("parallel",)),
        cost_estimate=pl.CostEstimate(flops=int(n * 4.2e6),
                                      transcendentals=int(n * 140_000),
                                      bytes_accessed=int(n * 4200)),
    )(xs, adjc, membc, w_slab)

    r = out.reshape(ng * _G, _B, _I, 8)[:n]
    logits = r[:, :, :, 0]
    values = r[:, :, :, 1]
    logp_all = r[:, :, :, 2]
    entropy = r[:, :, 0, 3]

    base = jax.random.key(sample_key[0])
    keys = jax.random.split(base, n)
    actions = jax.vmap(lambda ki, lg: jax.random.categorical(ki, lg, axis=-1))(keys, logits)
    log_probs = jnp.take_along_axis(logp_all, actions[:, :, None], axis=-1)[:, :, 0]
    return actions, log_probs, entropy, values


# speedup vs baseline: 4.6927x; 1.2442x over previous
import jax
import jax.numpy as jnp
from jax.experimental import pallas as pl
from jax.experimental.pallas import tpu as pltpu

_B = 2
_S = 16
_I = 4
_FIN = 8
_H = 32
_HEADS = 4
_G = 4
_NGR = _G * _B
_ROWS = _NGR * _S
_PROWS = _NGR * _I
_LANES = 128

_R_W1 = 0
_R_W2 = 8
_R_AD1 = 40
_R_AS1 = 48
_R_AD2 = 56
_R_AS2 = 64
_R_GB = 72
_R_WH1 = 80
_R_WH1B = 112
_R_WH2 = 120
_R_WH2B = 128
_WBLK = 136
_CH = 4


def _gat_layer(feat, w_all, a_dst_rows, a_src_rows, bias, mask):
    hf = jnp.dot(feat, w_all, preferred_element_type=jnp.float32)
    ed = jax.lax.dot_general(hf, a_dst_rows,
                             dimension_numbers=(((1,), (1,)), ((), ())),
                             preferred_element_type=jnp.float32)
    es = jax.lax.dot_general(a_src_rows, hf,
                             dimension_numbers=(((1,), (1,)), ((), ())),
                             preferred_element_type=jnp.float32)

    acc = None
    for h in range(_HEADS):
        e = ed[:, h:h + 1] + es[h:h + 1, :]
        e = jnp.where(e > 0, e, 0.2 * e) + mask
        e = e - jnp.max(e, axis=-1, keepdims=True)
        p = jnp.exp(e)
        denom = jnp.maximum(jnp.sum(p, axis=-1, keepdims=True), 1e-30)
        alpha = p / denom
        part = jnp.dot(alpha, hf[:, h * _H:(h + 1) * _H],
                       preferred_element_type=jnp.float32)
        acc = part if acc is None else acc + part
    return acc * (1.0 / _HEADS) + bias


def _fused_kernel(x_ref, adj_ref, memb_ref, w_ref, out_ref):
    f32 = jnp.float32
    w1 = w_ref[_R_W1:_R_W1 + _FIN, :]
    w2 = w_ref[_R_W2:_R_W2 + _H, :]
    ad1 = w_ref[_R_AD1:_R_AD1 + _HEADS, :]
    as1 = w_ref[_R_AS1:_R_AS1 + _HEADS, :]
    ad2 = w_ref[_R_AD2:_R_AD2 + _HEADS, :]
    as2 = w_ref[_R_AS2:_R_AS2 + _HEADS, :]
    b1 = w_ref[_R_GB:_R_GB + 1, 0:_H]
    b2 = w_ref[_R_GB + 1:_R_GB + 2, 0:_H]
    wh1_w = w_ref[_R_WH1:_R_WH1 + _H, 0:2 * _H]
    wh1_b = w_ref[_R_WH1B:_R_WH1B + 1, 0:2 * _H]
    wh2_wt = w_ref[_R_WH2:_R_WH2 + 2, 0:2 * _H]
    wh2_b = w_ref[_R_WH2B:_R_WH2B + 1, 0:2]

    row_g = jax.lax.broadcasted_iota(jnp.int32, (_ROWS, _ROWS), 0) // _S
    lane_g = jax.lax.broadcasted_iota(jnp.int32, (_ROWS, _ROWS), 1) // _S
    same_g = row_g == lane_g
    prow_g = jax.lax.broadcasted_iota(jnp.int32, (_PROWS, _ROWS), 0) // _I
    plane_g = jax.lax.broadcasted_iota(jnp.int32, (_PROWS, _ROWS), 1) // _S
    psame = prow_g == plane_g
    gr = jax.lax.broadcasted_iota(jnp.int32, (_PROWS, _PROWS), 0) // _I
    gc = jax.lax.broadcasted_iota(jnp.int32, (_PROWS, _PROWS), 1) // _I
    grp = (gr == gc).astype(f32)
    lane = jax.lax.broadcasted_iota(jnp.int32, (_PROWS, 8), 1)

    for ch in range(_CH):
        x = x_ref[ch]
        adjc = adj_ref[ch]
        membc = memb_ref[ch]

        adj_t = jnp.concatenate([adjc] * _NGR, axis=1)
        mask = jnp.where(same_g & (adj_t > 0), 0.0, -1e9)

        h1 = jnp.maximum(_gat_layer(x, w1, ad1, as1, b1, mask), 0.0)
        h2 = jnp.maximum(_gat_layer(h1, w2, ad2, as2, b2, mask), 0.0)

        cnt = jnp.maximum(jnp.sum(membc, axis=-1, keepdims=True), 1.0)
        memb_t = jnp.concatenate([membc / cnt] * _NGR, axis=1)
        memb_bd = jnp.where(psame, memb_t, 0.0)
        sp = jnp.dot(memb_bd, h2, preferred_element_type=f32)

        hid = jnp.maximum(jnp.dot(sp, wh1_w, preferred_element_type=f32) + wh1_b, 0.0)
        lv = jax.lax.dot_general(hid, wh2_wt,
                                 dimension_numbers=(((1,), (1,)), ((), ())),
                                 preferred_element_type=f32) + wh2_b

        logits_c = lv[:, 0:1]
        values_c = lv[:, 1:2]
        z = logits_c - jnp.max(logits_c)
        ez = jnp.exp(z)
        ssum = jnp.dot(grp, ez, preferred_element_type=f32)
        logp = z - jnp.log(ssum)
        pr = jnp.exp(logp)
        ent = -jnp.dot(grp, pr * logp, preferred_element_type=f32)

        out = (jnp.where(lane == 0, logits_c, 0.0)
               + jnp.where(lane == 1, values_c, 0.0)
               + jnp.where(lane == 2, logp, 0.0)
               + jnp.where(lane == 3, ent, 0.0))
        out_ref[ch] = out.astype(out_ref.dtype)


def kernel(x, adj, memb, w_slab, sample_key):
    n = x.shape[0]
    per_step = _G * _CH
    ng = -(-n // per_step) * _CH
    pad = ng * _G - n
    if pad:
        x = jnp.concatenate([x, jnp.zeros((pad,) + x.shape[1:], x.dtype)], 0)
        eye = jnp.broadcast_to(jnp.eye(_S, dtype=adj.dtype), (pad, _B, _S, _S))
        adj = jnp.concatenate([adj, eye], 0)
        memb = jnp.concatenate([memb, jnp.zeros((pad,) + memb.shape[1:], memb.dtype)], 0)

    xs = x.reshape(ng, _ROWS, _FIN)
    adjc = adj.reshape(ng, _ROWS, _S)
    membc = memb.reshape(ng, _PROWS, _S)

    out = pl.pallas_call(
        _fused_kernel,
        out_shape=jax.ShapeDtypeStruct((ng, _PROWS, 8), jnp.float32),
        grid=(ng // _CH,),
        in_specs=[
            pl.BlockSpec((_CH, _ROWS, _FIN), lambda g: (g, 0, 0)),
            pl.BlockSpec((_CH, _ROWS, _S), lambda g: (g, 0, 0)),
            pl.BlockSpec((_CH, _PROWS, _S), lambda g: (g, 0, 0)),
            pl.BlockSpec((_WBLK, _LANES), lambda g: (0, 0)),
        ],
        out_specs=pl.BlockSpec((_CH, _PROWS, 8), lambda g: (g, 0, 0)),
        compiler_params=pltpu.CompilerParams(dimension_semantics=("parallel",)),
        cost_estimate=pl.CostEstimate(flops=int(n * 4.2e6),
                                      transcendentals=int(n * 140_000),
                                      bytes_accessed=int(n * 4200)),
    )(xs, adjc, membc, w_slab)

    r = out.reshape(ng * _G, _B, _I, 8)[:n]
    logits = r[:, :, :, 0]
    values = r[:, :, :, 1]
    logp_all = r[:, :, :, 2]
    entropy = r[:, :, 0, 3]

    base = jax.random.key(sample_key[0])
    keys = jax.random.split(base, n)
    actions = jax.vmap(lambda ki, lg: jax.random.categorical(ki, lg, axis=-1))(keys, logits)
    log_probs = jnp.take_along_axis(logp_all, actions[:, :, None], axis=-1)[:, :, 0]
    return actions, log_probs, entropy, values
